# baseline (speedup 1.0000x reference)
import functools

import jax
import jax.numpy as jnp
from jax import lax
from jax.experimental import pallas as pl
from jax.experimental.pallas import tpu as pltpu
from jax.experimental.pallas import tpu_sc as plsc

N = 50000
D = 128
K = 16
HID = 16
C = 40
NP = 50400
RT = NP // 16
CW = 32
NCH = D // CW
NEP = 25088
EB = 8
NB = NEP // EB
PB1 = NB // 16
PB2 = NB // 32
INV_KM1 = 1.0 / (K - 1)
BR = 504
BF = 400


def _nsqrt(x):
    xi = plsc.bitcast(x, jnp.int32)
    y = plsc.bitcast(jnp.int32(0x5F3759DF) - (xi >> 1), jnp.float32)
    xh = 0.5 * x
    y = y * (1.5 - xh * y * y)
    y = y * (1.5 - xh * y * y)
    return x * y


SB = 63
NSB = RT // SB


def _sc1_body(H, en, out, hc, hp, acc, hbuf_a, hbuf_b,
              idxr_q0, idxr_q1, idxr_q2, idxr_q3,
              idxo_a, idxo_b, rows_a, rows_b, contrib,
              psem_a, psem_b, osem_a, osem_b,
              sem_i0, sem_i1, sem_i2, sem_i3, sem_a, sem_b):
    c = lax.axis_index("c")
    s = lax.axis_index("s")

    z16 = jnp.zeros((16,), jnp.float32)

    col0 = c * (2 * CW)
    row_t = s * RT

    def p0_start(i, buf, sem):
        @pl.when(i < NSB)
        def _():
            r0 = jnp.minimum(row_t + i * SB, N - SB)
            pltpu.async_copy(H.at[pl.ds(r0, SB), pl.ds(col0, 2 * CW)], buf, sem)

    def p0_outs(i, buf):
        r0 = jnp.minimum(row_t + i * SB, N - SB)
        for j in range(2):
            chunk = c * 2 + j
            src = buf.at[pl.ds(0, SB), pl.ds(j * CW, CW)]
            yield (src, hc.at[pl.ds(chunk * NP + r0, SB)])
            yield (src, hp.at[pl.ds(chunk * NP + r0, SB)])

    def p0_proc(i, buf, psem, osem):
        pltpu.make_async_copy(H.at[pl.ds(0, SB), pl.ds(col0, 2 * CW)],
                              buf, psem).wait()
        @pl.when(i >= 2)
        def _():
            for src, dst in list(p0_outs(i, buf))[1::2]:
                pltpu.make_async_copy(src, dst, osem).wait()

        def rowclip(r, _):
            for v in range(4):
                sl = pl.ds(v * 16, 16)
                buf[r, sl] = jnp.clip(buf[r, sl], 1e-7, 10.0)
            return 0

        lax.fori_loop(0, SB, rowclip, 0)
        srcdst = list(p0_outs(i, buf))
        for src, dst in srcdst[0::2]:
            pltpu.async_copy(src, dst, osem)
        for src, dst in srcdst[0::2]:
            pltpu.make_async_copy(src, dst, osem).wait()

        def rowsq(r, _):
            for v in range(4):
                sl = pl.ds(v * 16, 16)
                x = buf[r, sl]
                buf[r, sl] = x * x
            return 0

        lax.fori_loop(0, SB, rowsq, 0)
        for src, dst in srcdst[1::2]:
            pltpu.async_copy(src, dst, osem)

    p0_start(0, hbuf_a, psem_a)

    def p0_pair(it, _):
        i = it * 2
        p0_start(i + 1, hbuf_b, psem_b)
        p0_proc(i, hbuf_a, psem_a, osem_a)
        p0_start(i + 2, hbuf_a, psem_a)
        p0_proc(i + 1, hbuf_b, psem_b, osem_b)
        return 0

    lax.fori_loop(0, NSB // 2, p0_pair, 0)
    for buf, osem, i in ((hbuf_a, osem_a, NSB - 2), (hbuf_b, osem_b, NSB - 1)):
        for src, dst in list(p0_outs(i, buf))[1::2]:
            pltpu.make_async_copy(src, dst, osem).wait()
    plsc.subcore_barrier()

    def idx_start(i, q, isem):
        b = i * 16 + s
        pltpu.async_copy(en.at[pl.ds(b * 128, 128)], q, isem)

    def mk_off(q, isem, idxo, base):
        pltpu.make_async_copy(en.at[pl.ds(0, 128)], q, isem).wait()
        for v in range(8):
            sl = pl.ds(v * 16, 16)
            idxo[sl] = q[sl] + base

    def compute_scatter(idxr, rows):
        for e in range(EB):
            r0 = e * K

            def ksum(kk, tt):
                r = r0 + kk * 2
                a0 = tt[0] + rows[r, pl.ds(0, 16)] + rows[r + 1, pl.ds(0, 16)]
                a1 = tt[1] + rows[r, pl.ds(16, 16)] + rows[r + 1, pl.ds(16, 16)]
                return (a0, a1)

            t0, t1 = lax.fori_loop(0, K // 2, ksum, (z16, z16))

            def kcon(kk, _):
                for u in range(2):
                    r = r0 + kk * 2 + u
                    contrib[r, pl.ds(0, 16)] = _nsqrt(
                        (t0 - rows[r, pl.ds(0, 16)]) * INV_KM1)
                    contrib[r, pl.ds(16, 16)] = _nsqrt(
                        (t1 - rows[r, pl.ds(16, 16)]) * INV_KM1)
                return 0

            lax.fori_loop(0, K // 2, kcon, 0)
        pltpu.sync_copy(contrib, acc.at[idxr], add=True)

    def fetch(i, idxr, isem, idxo, rows, sem, base):
        idx_start(i, idxr, isem)
        mk_off(idxr, isem, idxo, base)
        pltpu.async_copy(hp.at[idxo], rows, sem)

    def chunk_pass(j, _):
        chunk = c * 2 + j
        base = chunk * NP
        pltpu.sync_copy(hc.at[pl.ds(base + s * RT, RT)], acc.at[pl.ds(s * RT, RT)])
        fetch(0, idxr_q0, sem_i0, idxo_a, rows_a, sem_a, base)
        plsc.subcore_barrier()

        def pair(it, _):
            i = it * 2
            fetch(i + 1, idxr_q1, sem_i1, idxo_b, rows_b, sem_b, base)
            pltpu.make_async_copy(hp.at[idxo_a], rows_a, sem_a).wait()
            compute_scatter(idxr_q0, rows_a)

            @pl.when(it + 1 < PB1 // 2)
            def _():
                fetch(i + 2, idxr_q0, sem_i0, idxo_a, rows_a, sem_a, base)

            pltpu.make_async_copy(hp.at[idxo_b], rows_b, sem_b).wait()
            compute_scatter(idxr_q1, rows_b)
            return 0

        lax.fori_loop(0, PB1 // 2, pair, 0)
        plsc.subcore_barrier()
        pltpu.sync_copy(acc.at[pl.ds(s * RT, RT)],
                        out.at[pl.ds(s * RT, RT), pl.ds(chunk * CW, CW)])
        plsc.subcore_barrier()
        return 0

    lax.fori_loop(0, 2, chunk_pass, 0)


def _sc2_body(h1p, h1ch, en, out, acc, idxr_a, idxr_b, rows_a, rows_b, contrib,
              sem_a, sem_b):
    c = lax.axis_index("c")
    s = lax.axis_index("s")

    z16 = jnp.zeros((16,), jnp.float32)

    def fetch(i, idxr, rows, sem):
        b = c * (NB // 2) + i * 16 + s
        pltpu.sync_copy(en.at[pl.ds(b * 128, 128)], idxr)
        pltpu.async_copy(h1p.at[idxr], rows, sem)

    def compute_scatter(idxr, rows):
        for e in range(EB):
            r0 = e * K

            def ksum(kk, a):
                r = r0 + kk * 2
                return a + rows[r, :] + rows[r + 1, :]

            t = lax.fori_loop(0, K // 2, ksum, z16)

            def kcon(kk, _):
                for u in range(2):
                    r = r0 + kk * 2 + u
                    contrib[r, :] = _nsqrt((t - rows[r, :]) * INV_KM1)
                return 0

            lax.fori_loop(0, K // 2, kcon, 0)
        pltpu.sync_copy(contrib, acc.at[idxr], add=True)

    fetch(0, idxr_a, rows_a, sem_a)
    pltpu.sync_copy(h1ch.at[pl.ds(s * RT, RT)], acc.at[pl.ds(s * RT, RT)])
    plsc.subcore_barrier()

    def pair(it, _):
        i = it * 2
        fetch(i + 1, idxr_b, rows_b, sem_b)
        pltpu.make_async_copy(h1p.at[idxr_a], rows_a, sem_a).wait()
        compute_scatter(idxr_a, rows_a)

        @pl.when(it + 1 < PB2 // 2)
        def _():
            fetch(i + 2, idxr_a, rows_a, sem_a)

        pltpu.make_async_copy(h1p.at[idxr_b], rows_b, sem_b).wait()
        compute_scatter(idxr_b, rows_b)
        return 0

    lax.fori_loop(0, PB2 // 2, pair, 0)
    plsc.subcore_barrier()
    pltpu.sync_copy(acc.at[pl.ds(s * RT, RT)], out.at[pl.ds(c * NP + s * RT, RT)])


def _mid_body(l1_ref, w1_ref, b1_ref, h1p_ref, h1ch_ref):
    x = l1_ref[...]
    rs = jnp.sum(x, axis=1)
    h = jnp.dot(x, w1_ref[...], preferred_element_type=jnp.float32)
    rinv = 1.0 / rs
    rinv = jnp.where(jnp.isinf(rinv), 0.0, rinv)
    h1 = jnp.maximum(h * rinv[:, None] + b1_ref[...], 0.0)
    h1c = jnp.clip(h1, 1e-7, 10.0)
    h1p_ref[...] = h1c * h1c
    h1ch_ref[...] = 0.5 * h1c


def _fin_body(p0_ref, p1_ref, w2_ref, b2_ref, out_ref):
    ah = p0_ref[...] + p1_ref[...]
    rs = jnp.sum(ah, axis=1)
    rinv = 1.0 / rs
    rinv = jnp.where(jnp.isinf(rinv), 0.0, rinv)
    out_ref[...] = (jnp.dot(ah, w2_ref[...], preferred_element_type=jnp.float32)
                    * rinv[:, None] + b2_ref[...])


@functools.lru_cache(maxsize=None)
def _sc_kernels():
    mesh = plsc.VectorSubcoreMesh(
        core_axis_name="c", subcore_axis_name="s", num_cores=2, num_subcores=16)
    params = pltpu.CompilerParams(
        needs_layout_passes=False, use_tc_tiling_on_sc=False)
    sc1 = pl.kernel(
        _sc1_body,
        out_type=(jax.ShapeDtypeStruct((NP, D), jnp.float32),
                  jax.ShapeDtypeStruct((NCH * NP, CW), jnp.float32),
                  jax.ShapeDtypeStruct((NCH * NP, CW), jnp.float32)),
        mesh=mesh,
        compiler_params=params,
        scratch_types=(
            [pltpu.VMEM_SHARED((NP, CW), jnp.float32)] +
            [pltpu.VMEM((SB, 2 * CW), jnp.float32)] * 2 +
            [pltpu.VMEM((128,), jnp.int32)] * 6 +
            [pltpu.VMEM((128, CW), jnp.float32)] * 3 +
            [pltpu.SemaphoreType.DMA] * 10))
    sc2 = pl.kernel(
        _sc2_body,
        out_type=jax.ShapeDtypeStruct((2 * NP, HID), jnp.float32),
        mesh=mesh,
        compiler_params=params,
        scratch_types=[
            pltpu.VMEM_SHARED((NP, HID), jnp.float32),
            pltpu.VMEM((128,), jnp.int32),
            pltpu.VMEM((128,), jnp.int32),
            pltpu.VMEM((128, HID), jnp.float32),
            pltpu.VMEM((128, HID), jnp.float32),
            pltpu.VMEM((128, HID), jnp.float32),
            pltpu.SemaphoreType.DMA,
            pltpu.SemaphoreType.DMA,
        ])
    return sc1, sc2


def kernel(H, edge_nodes, W1, b1, W2, b2):
    f32 = jnp.float32
    sc1, sc2 = _sc_kernels()
    ne = edge_nodes.shape[0]
    en = jnp.concatenate(
        [edge_nodes.astype(jnp.int32),
         jnp.full((NEP - ne, K), N, jnp.int32)], axis=0).reshape(-1)

    nrb = NP // BR
    l1, _hc, _hp = sc1(H.astype(f32), en)

    h1p, h1ch = pl.pallas_call(
        _mid_body,
        grid=(nrb,),
        in_specs=[pl.BlockSpec((BR, D), lambda i: (i, 0)),
                  pl.BlockSpec((D, HID), lambda i: (0, 0)),
                  pl.BlockSpec((1, HID), lambda i: (0, 0))],
        out_specs=[pl.BlockSpec((BR, HID), lambda i: (i, 0))] * 2,
        out_shape=[jax.ShapeDtypeStruct((NP, HID), f32)] * 2,
    )(l1, W1.astype(f32), b1.astype(f32).reshape(1, HID))

    l2 = sc2(h1p, h1ch, en)

    npb = NP // BF
    out = pl.pallas_call(
        _fin_body,
        grid=(N // BF,),
        in_specs=[pl.BlockSpec((BF, HID), lambda i, c=c: (c * npb + i, 0))
                  for c in range(2)] +
                 [pl.BlockSpec((HID, C), lambda i: (0, 0)),
                  pl.BlockSpec((1, C), lambda i: (0, 0))],
        out_specs=pl.BlockSpec((BF, C), lambda i: (i, 0)),
        out_shape=jax.ShapeDtypeStruct((N, C), f32),
    )(l2, l2, W2.astype(f32), b2.astype(f32).reshape(1, C))
    return out


# speedup vs baseline: 1.1881x; 1.1881x over previous
import functools

import jax
import jax.numpy as jnp
from jax import lax
from jax.experimental import pallas as pl
from jax.experimental.pallas import tpu as pltpu
from jax.experimental.pallas import tpu_sc as plsc

N = 50000
D = 128
K = 16
HID = 16
C = 40
NP = 50400
RT = NP // 16
CW = 32
NCH = D // CW
NEP = 25088
EB = 8
NB = NEP // EB
PB1 = NB // 16
PB2 = NB // 32
INV_KM1 = 1.0 / (K - 1)
BR = 504
BF = 400


def _nsqrt(x):
    xi = plsc.bitcast(x, jnp.int32)
    y = plsc.bitcast(jnp.int32(0x5F3759DF) - (xi >> 1), jnp.float32)
    xh = 0.5 * x
    y = y * (1.5 - xh * y * y)
    y = y * (1.5 - xh * y * y)
    return x * y


SB = 63
NSB = RT // SB


def _sc1_body(H, en, out, hc, hp, acc, hbuf_a, hbuf_b,
              idxr_q0, idxr_q1, idxr_q2, idxr_q3,
              idxo_a, idxo_b, rows_a, rows_b, contrib,
              psem_a, psem_b, osem_a, osem_b,
              sem_i0, sem_i1, sem_i2, sem_i3, sem_a, sem_b):
    c = lax.axis_index("c")
    s = lax.axis_index("s")

    z16 = jnp.zeros((16,), jnp.float32)

    col0 = c * (2 * CW)
    row_t = s * RT

    def p0_start(i, buf, sem):
        @pl.when(i < NSB)
        def _():
            r0 = jnp.minimum(row_t + i * SB, N - SB)
            pltpu.async_copy(H.at[pl.ds(r0, SB), pl.ds(col0, 2 * CW)], buf, sem)

    def p0_outs(i, buf):
        r0 = jnp.minimum(row_t + i * SB, N - SB)
        for j in range(2):
            chunk = c * 2 + j
            src = buf.at[pl.ds(0, SB), pl.ds(j * CW, CW)]
            yield (src, hc.at[pl.ds(chunk * NP + r0, SB)])
            yield (src, hp.at[pl.ds(chunk * NP + r0, SB)])

    def p0_proc(i, buf, psem, osem):
        pltpu.make_async_copy(H.at[pl.ds(0, SB), pl.ds(col0, 2 * CW)],
                              buf, psem).wait()
        @pl.when(i >= 2)
        def _():
            for src, dst in list(p0_outs(i, buf))[1::2]:
                pltpu.make_async_copy(src, dst, osem).wait()

        def rowclip(r, _):
            for v in range(4):
                sl = pl.ds(v * 16, 16)
                buf[r, sl] = jnp.clip(buf[r, sl], 1e-7, 10.0)
            return 0

        lax.fori_loop(0, SB, rowclip, 0)
        srcdst = list(p0_outs(i, buf))
        for src, dst in srcdst[0::2]:
            pltpu.async_copy(src, dst, osem)
        for src, dst in srcdst[0::2]:
            pltpu.make_async_copy(src, dst, osem).wait()

        def rowsq(r, _):
            for v in range(4):
                sl = pl.ds(v * 16, 16)
                x = buf[r, sl]
                buf[r, sl] = x * x
            return 0

        lax.fori_loop(0, SB, rowsq, 0)
        for src, dst in srcdst[1::2]:
            pltpu.async_copy(src, dst, osem)

    p0_start(0, hbuf_a, psem_a)

    def p0_pair(it, _):
        i = it * 2
        p0_start(i + 1, hbuf_b, psem_b)
        p0_proc(i, hbuf_a, psem_a, osem_a)
        p0_start(i + 2, hbuf_a, psem_a)
        p0_proc(i + 1, hbuf_b, psem_b, osem_b)
        return 0

    lax.fori_loop(0, NSB // 2, p0_pair, 0)
    for buf, osem, i in ((hbuf_a, osem_a, NSB - 2), (hbuf_b, osem_b, NSB - 1)):
        for src, dst in list(p0_outs(i, buf))[1::2]:
            pltpu.make_async_copy(src, dst, osem).wait()
    plsc.subcore_barrier()

    def idx_start(i, q, isem):
        b = i * 16 + s
        pltpu.async_copy(en.at[pl.ds(b * 128, 128)], q, isem)

    def mk_off(q, isem, idxo, base):
        pltpu.make_async_copy(en.at[pl.ds(0, 128)], q, isem).wait()
        for v in range(8):
            sl = pl.ds(v * 16, 16)
            idxo[sl] = q[sl] + base

    def compute_scatter(idxr, rows):
        for e in range(EB):
            r0 = e * K

            def ksum(kk, tt):
                return (tt[0] + rows[r0 + kk, pl.ds(0, 16)],
                        tt[1] + rows[r0 + kk, pl.ds(16, 16)])

            t0, t1 = lax.fori_loop(0, K, ksum, (z16, z16))

            def kcon(kk, _):
                contrib[r0 + kk, pl.ds(0, 16)] = _nsqrt(
                    (t0 - rows[r0 + kk, pl.ds(0, 16)]) * INV_KM1)
                contrib[r0 + kk, pl.ds(16, 16)] = _nsqrt(
                    (t1 - rows[r0 + kk, pl.ds(16, 16)]) * INV_KM1)
                return 0

            lax.fori_loop(0, K, kcon, 0)
        pltpu.sync_copy(contrib, acc.at[idxr], add=True)

    def fetch(i, idxr, isem, idxo, rows, sem, base):
        idx_start(i, idxr, isem)
        mk_off(idxr, isem, idxo, base)
        pltpu.async_copy(hp.at[idxo], rows, sem)

    def chunk_pass(j, _):
        chunk = c * 2 + j
        base = chunk * NP
        pltpu.sync_copy(hc.at[pl.ds(base + s * RT, RT)], acc.at[pl.ds(s * RT, RT)])
        fetch(0, idxr_q0, sem_i0, idxo_a, rows_a, sem_a, base)
        plsc.subcore_barrier()

        def pair(it, _):
            i = it * 2
            fetch(i + 1, idxr_q1, sem_i1, idxo_b, rows_b, sem_b, base)
            pltpu.make_async_copy(hp.at[idxo_a], rows_a, sem_a).wait()
            compute_scatter(idxr_q0, rows_a)

            @pl.when(it + 1 < PB1 // 2)
            def _():
                fetch(i + 2, idxr_q0, sem_i0, idxo_a, rows_a, sem_a, base)

            pltpu.make_async_copy(hp.at[idxo_b], rows_b, sem_b).wait()
            compute_scatter(idxr_q1, rows_b)
            return 0

        lax.fori_loop(0, PB1 // 2, pair, 0)
        plsc.subcore_barrier()
        pltpu.sync_copy(acc.at[pl.ds(s * RT, RT)],
                        out.at[pl.ds(s * RT, RT), pl.ds(chunk * CW, CW)])
        plsc.subcore_barrier()
        return 0

    lax.fori_loop(0, 2, chunk_pass, 0)


def _sc2_body(h1p, h1ch, en, out, acc, idxr_a, idxr_b, rows_a, rows_b, contrib,
              sem_a, sem_b):
    c = lax.axis_index("c")
    s = lax.axis_index("s")

    z16 = jnp.zeros((16,), jnp.float32)

    def fetch(i, idxr, rows, sem):
        b = c * (NB // 2) + i * 16 + s
        pltpu.sync_copy(en.at[pl.ds(b * 128, 128)], idxr)
        pltpu.async_copy(h1p.at[idxr], rows, sem)

    def compute_scatter(idxr, rows):
        for e in range(EB):
            r0 = e * K

            def ksum(kk, a):
                return a + rows[r0 + kk, :]

            t = lax.fori_loop(0, K, ksum, z16)

            def kcon(kk, _):
                contrib[r0 + kk, :] = _nsqrt((t - rows[r0 + kk, :]) * INV_KM1)
                return 0

            lax.fori_loop(0, K, kcon, 0)
        pltpu.sync_copy(contrib, acc.at[idxr], add=True)

    fetch(0, idxr_a, rows_a, sem_a)
    pltpu.sync_copy(h1ch.at[pl.ds(s * RT, RT)], acc.at[pl.ds(s * RT, RT)])
    plsc.subcore_barrier()

    def pair(it, _):
        i = it * 2
        fetch(i + 1, idxr_b, rows_b, sem_b)
        pltpu.make_async_copy(h1p.at[idxr_a], rows_a, sem_a).wait()
        compute_scatter(idxr_a, rows_a)

        @pl.when(it + 1 < PB2 // 2)
        def _():
            fetch(i + 2, idxr_a, rows_a, sem_a)

        pltpu.make_async_copy(h1p.at[idxr_b], rows_b, sem_b).wait()
        compute_scatter(idxr_b, rows_b)
        return 0

    lax.fori_loop(0, PB2 // 2, pair, 0)
    plsc.subcore_barrier()
    pltpu.sync_copy(acc.at[pl.ds(s * RT, RT)], out.at[pl.ds(c * NP + s * RT, RT)])


def _mid_body(l1_ref, w1_ref, b1_ref, h1p_ref, h1ch_ref):
    x = l1_ref[...]
    rs = jnp.sum(x, axis=1)
    h = jnp.dot(x, w1_ref[...], preferred_element_type=jnp.float32)
    rinv = 1.0 / rs
    rinv = jnp.where(jnp.isinf(rinv), 0.0, rinv)
    h1 = jnp.maximum(h * rinv[:, None] + b1_ref[...], 0.0)
    h1c = jnp.clip(h1, 1e-7, 10.0)
    h1p_ref[...] = h1c * h1c
    h1ch_ref[...] = 0.5 * h1c


def _fin_body(p0_ref, p1_ref, w2_ref, b2_ref, out_ref):
    ah = p0_ref[...] + p1_ref[...]
    rs = jnp.sum(ah, axis=1)
    rinv = 1.0 / rs
    rinv = jnp.where(jnp.isinf(rinv), 0.0, rinv)
    out_ref[...] = (jnp.dot(ah, w2_ref[...], preferred_element_type=jnp.float32)
                    * rinv[:, None] + b2_ref[...])


@functools.lru_cache(maxsize=None)
def _sc_kernels():
    mesh = plsc.VectorSubcoreMesh(
        core_axis_name="c", subcore_axis_name="s", num_cores=2, num_subcores=16)
    params = pltpu.CompilerParams(
        needs_layout_passes=False, use_tc_tiling_on_sc=False)
    sc1 = pl.kernel(
        _sc1_body,
        out_type=(jax.ShapeDtypeStruct((NP, D), jnp.float32),
                  jax.ShapeDtypeStruct((NCH * NP, CW), jnp.float32),
                  jax.ShapeDtypeStruct((NCH * NP, CW), jnp.float32)),
        mesh=mesh,
        compiler_params=params,
        scratch_types=(
            [pltpu.VMEM_SHARED((NP, CW), jnp.float32)] +
            [pltpu.VMEM((SB, 2 * CW), jnp.float32)] * 2 +
            [pltpu.VMEM((128,), jnp.int32)] * 6 +
            [pltpu.VMEM((128, CW), jnp.float32)] * 3 +
            [pltpu.SemaphoreType.DMA] * 10))
    sc2 = pl.kernel(
        _sc2_body,
        out_type=jax.ShapeDtypeStruct((2 * NP, HID), jnp.float32),
        mesh=mesh,
        compiler_params=params,
        scratch_types=[
            pltpu.VMEM_SHARED((NP, HID), jnp.float32),
            pltpu.VMEM((128,), jnp.int32),
            pltpu.VMEM((128,), jnp.int32),
            pltpu.VMEM((128, HID), jnp.float32),
            pltpu.VMEM((128, HID), jnp.float32),
            pltpu.VMEM((128, HID), jnp.float32),
            pltpu.SemaphoreType.DMA,
            pltpu.SemaphoreType.DMA,
        ])
    return sc1, sc2


def kernel(H, edge_nodes, W1, b1, W2, b2):
    f32 = jnp.float32
    sc1, sc2 = _sc_kernels()
    ne = edge_nodes.shape[0]
    en = jnp.concatenate(
        [edge_nodes.astype(jnp.int32),
         jnp.full((NEP - ne, K), N, jnp.int32)], axis=0).reshape(-1)

    nrb = NP // BR
    l1, _hc, _hp = sc1(H.astype(f32), en)

    h1p, h1ch = pl.pallas_call(
        _mid_body,
        grid=(nrb,),
        in_specs=[pl.BlockSpec((BR, D), lambda i: (i, 0)),
                  pl.BlockSpec((D, HID), lambda i: (0, 0)),
                  pl.BlockSpec((1, HID), lambda i: (0, 0))],
        out_specs=[pl.BlockSpec((BR, HID), lambda i: (i, 0))] * 2,
        out_shape=[jax.ShapeDtypeStruct((NP, HID), f32)] * 2,
    )(l1, W1.astype(f32), b1.astype(f32).reshape(1, HID))

    l2 = sc2(h1p, h1ch, en)

    npb = NP // BF
    out = pl.pallas_call(
        _fin_body,
        grid=(N // BF,),
        in_specs=[pl.BlockSpec((BF, HID), lambda i, c=c: (c * npb + i, 0))
                  for c in range(2)] +
                 [pl.BlockSpec((HID, C), lambda i: (0, 0)),
                  pl.BlockSpec((1, C), lambda i: (0, 0))],
        out_specs=pl.BlockSpec((BF, C), lambda i: (i, 0)),
        out_shape=jax.ShapeDtypeStruct((N, C), f32),
    )(l2, l2, W2.astype(f32), b2.astype(f32).reshape(1, C))
    return out


# speedup vs baseline: 1.3801x; 1.1615x over previous
import functools

import jax
import jax.numpy as jnp
from jax import lax
from jax.experimental import pallas as pl
from jax.experimental.pallas import tpu as pltpu
from jax.experimental.pallas import tpu_sc as plsc

N = 50000
D = 128
K = 16
HID = 16
C = 40
NP = 50400
RT = NP // 16
CW = 32
NCH = D // CW
NEP = 25088
EB = 8
NB = NEP // EB
PB1 = NB // 16
PB2 = NB // 32
INV_KM1 = 1.0 / (K - 1)
BR = 504
BF = 400


def _nsqrt(x):
    xi = plsc.bitcast(x, jnp.int32)
    y = plsc.bitcast(jnp.int32(0x5F3759DF) - (xi >> 1), jnp.float32)
    xh = 0.5 * x
    y = y * (1.5 - xh * y * y)
    return x * y


SB = 63
NSB = RT // SB


def _sc1_body(H, en, out, hc, hp, acc, hbuf_a, hbuf_b,
              idxr_q0, idxr_q1, idxr_q2, idxr_q3,
              idxo_a, idxo_b, rows_a, rows_b, contrib,
              psem_a, psem_b, osem_a, osem_b,
              sem_i0, sem_i1, sem_i2, sem_i3, sem_a, sem_b):
    c = lax.axis_index("c")
    s = lax.axis_index("s")

    z16 = jnp.zeros((16,), jnp.float32)

    col0 = c * (2 * CW)
    row_t = s * RT

    def p0_start(i, buf, sem):
        @pl.when(i < NSB)
        def _():
            r0 = jnp.minimum(row_t + i * SB, N - SB)
            pltpu.async_copy(H.at[pl.ds(r0, SB), pl.ds(col0, 2 * CW)], buf, sem)

    def p0_outs(i, buf):
        r0 = jnp.minimum(row_t + i * SB, N - SB)
        for j in range(2):
            chunk = c * 2 + j
            src = buf.at[pl.ds(0, SB), pl.ds(j * CW, CW)]
            yield (src, hc.at[pl.ds(chunk * NP + r0, SB)])
            yield (src, hp.at[pl.ds(chunk * NP + r0, SB)])

    def p0_proc(i, buf, psem, osem):
        pltpu.make_async_copy(H.at[pl.ds(0, SB), pl.ds(col0, 2 * CW)],
                              buf, psem).wait()
        @pl.when(i >= 2)
        def _():
            for src, dst in list(p0_outs(i, buf))[1::2]:
                pltpu.make_async_copy(src, dst, osem).wait()

        def rowclip(r, _):
            for v in range(4):
                sl = pl.ds(v * 16, 16)
                buf[r, sl] = jnp.clip(buf[r, sl], 1e-7, 10.0)
            return 0

        lax.fori_loop(0, SB, rowclip, 0)
        srcdst = list(p0_outs(i, buf))
        for src, dst in srcdst[0::2]:
            pltpu.async_copy(src, dst, osem)
        for src, dst in srcdst[0::2]:
            pltpu.make_async_copy(src, dst, osem).wait()

        def rowsq(r, _):
            for v in range(4):
                sl = pl.ds(v * 16, 16)
                x = buf[r, sl]
                buf[r, sl] = x * x
            return 0

        lax.fori_loop(0, SB, rowsq, 0)
        for src, dst in srcdst[1::2]:
            pltpu.async_copy(src, dst, osem)

    p0_start(0, hbuf_a, psem_a)

    def p0_pair(it, _):
        i = it * 2
        p0_start(i + 1, hbuf_b, psem_b)
        p0_proc(i, hbuf_a, psem_a, osem_a)
        p0_start(i + 2, hbuf_a, psem_a)
        p0_proc(i + 1, hbuf_b, psem_b, osem_b)
        return 0

    lax.fori_loop(0, NSB // 2, p0_pair, 0)
    for buf, osem, i in ((hbuf_a, osem_a, NSB - 2), (hbuf_b, osem_b, NSB - 1)):
        for src, dst in list(p0_outs(i, buf))[1::2]:
            pltpu.make_async_copy(src, dst, osem).wait()
    plsc.subcore_barrier()

    def idx_start(i, q, isem):
        b = i * 16 + s
        pltpu.async_copy(en.at[pl.ds(b * 128, 128)], q, isem)

    def mk_off(q, isem, idxo, base):
        pltpu.make_async_copy(en.at[pl.ds(0, 128)], q, isem).wait()
        for v in range(8):
            sl = pl.ds(v * 16, 16)
            idxo[sl] = q[sl] + base

    def compute_scatter(idxr, rows):
        for e in range(EB):
            r0 = e * K

            def ksum(kk, tt):
                return (tt[0] + rows[r0 + kk, pl.ds(0, 16)],
                        tt[1] + rows[r0 + kk, pl.ds(16, 16)])

            t0, t1 = lax.fori_loop(0, K, ksum, (z16, z16))

            def kcon(kk, _):
                contrib[r0 + kk, pl.ds(0, 16)] = _nsqrt(
                    (t0 - rows[r0 + kk, pl.ds(0, 16)]) * INV_KM1)
                contrib[r0 + kk, pl.ds(16, 16)] = _nsqrt(
                    (t1 - rows[r0 + kk, pl.ds(16, 16)]) * INV_KM1)
                return 0

            lax.fori_loop(0, K, kcon, 0)
        pltpu.sync_copy(contrib, acc.at[idxr], add=True)

    def fetch(i, idxr, isem, idxo, rows, sem, base):
        idx_start(i, idxr, isem)
        mk_off(idxr, isem, idxo, base)
        pltpu.async_copy(hp.at[idxo], rows, sem)

    def chunk_pass(j, _):
        chunk = c * 2 + j
        base = chunk * NP
        pltpu.sync_copy(hc.at[pl.ds(base + s * RT, RT)], acc.at[pl.ds(s * RT, RT)])
        fetch(0, idxr_q0, sem_i0, idxo_a, rows_a, sem_a, base)
        plsc.subcore_barrier()

        def pair(it, _):
            i = it * 2
            fetch(i + 1, idxr_q1, sem_i1, idxo_b, rows_b, sem_b, base)
            pltpu.make_async_copy(hp.at[idxo_a], rows_a, sem_a).wait()
            compute_scatter(idxr_q0, rows_a)

            @pl.when(it + 1 < PB1 // 2)
            def _():
                fetch(i + 2, idxr_q0, sem_i0, idxo_a, rows_a, sem_a, base)

            pltpu.make_async_copy(hp.at[idxo_b], rows_b, sem_b).wait()
            compute_scatter(idxr_q1, rows_b)
            return 0

        lax.fori_loop(0, PB1 // 2, pair, 0)
        plsc.subcore_barrier()
        pltpu.sync_copy(acc.at[pl.ds(s * RT, RT)],
                        out.at[pl.ds(s * RT, RT), pl.ds(chunk * CW, CW)])
        plsc.subcore_barrier()
        return 0

    lax.fori_loop(0, 2, chunk_pass, 0)


def _sc2_body(h1p, h1ch, en, out, acc, idxr_a, idxr_b, rows_a, rows_b, contrib,
              sem_a, sem_b):
    c = lax.axis_index("c")
    s = lax.axis_index("s")

    z16 = jnp.zeros((16,), jnp.float32)

    def fetch(i, idxr, rows, sem):
        b = c * (NB // 2) + i * 16 + s
        pltpu.sync_copy(en.at[pl.ds(b * 128, 128)], idxr)
        pltpu.async_copy(h1p.at[idxr], rows, sem)

    def compute_scatter(idxr, rows):
        for e in range(EB):
            r0 = e * K

            def ksum(kk, a):
                return a + rows[r0 + kk, :]

            t = lax.fori_loop(0, K, ksum, z16)

            def kcon(kk, _):
                contrib[r0 + kk, :] = _nsqrt((t - rows[r0 + kk, :]) * INV_KM1)
                return 0

            lax.fori_loop(0, K, kcon, 0)
        pltpu.sync_copy(contrib, acc.at[idxr], add=True)

    fetch(0, idxr_a, rows_a, sem_a)
    pltpu.sync_copy(h1ch.at[pl.ds(s * RT, RT)], acc.at[pl.ds(s * RT, RT)])
    plsc.subcore_barrier()

    def pair(it, _):
        i = it * 2
        fetch(i + 1, idxr_b, rows_b, sem_b)
        pltpu.make_async_copy(h1p.at[idxr_a], rows_a, sem_a).wait()
        compute_scatter(idxr_a, rows_a)

        @pl.when(it + 1 < PB2 // 2)
        def _():
            fetch(i + 2, idxr_a, rows_a, sem_a)

        pltpu.make_async_copy(h1p.at[idxr_b], rows_b, sem_b).wait()
        compute_scatter(idxr_b, rows_b)
        return 0

    lax.fori_loop(0, PB2 // 2, pair, 0)
    plsc.subcore_barrier()
    pltpu.sync_copy(acc.at[pl.ds(s * RT, RT)], out.at[pl.ds(c * NP + s * RT, RT)])


def _mid_body(l1_ref, w1_ref, b1_ref, h1p_ref, h1ch_ref):
    x = l1_ref[...]
    rs = jnp.sum(x, axis=1)
    h = jnp.dot(x, w1_ref[...], preferred_element_type=jnp.float32)
    rinv = 1.0 / rs
    rinv = jnp.where(jnp.isinf(rinv), 0.0, rinv)
    h1 = jnp.maximum(h * rinv[:, None] + b1_ref[...], 0.0)
    h1c = jnp.clip(h1, 1e-7, 10.0)
    h1p_ref[...] = h1c * h1c
    h1ch_ref[...] = 0.5 * h1c


def _fin_body(p0_ref, p1_ref, w2_ref, b2_ref, out_ref):
    ah = p0_ref[...] + p1_ref[...]
    rs = jnp.sum(ah, axis=1)
    rinv = 1.0 / rs
    rinv = jnp.where(jnp.isinf(rinv), 0.0, rinv)
    out_ref[...] = (jnp.dot(ah, w2_ref[...], preferred_element_type=jnp.float32)
                    * rinv[:, None] + b2_ref[...])


@functools.lru_cache(maxsize=None)
def _sc_kernels():
    mesh = plsc.VectorSubcoreMesh(
        core_axis_name="c", subcore_axis_name="s", num_cores=2, num_subcores=16)
    params = pltpu.CompilerParams(
        needs_layout_passes=False, use_tc_tiling_on_sc=False)
    sc1 = pl.kernel(
        _sc1_body,
        out_type=(jax.ShapeDtypeStruct((NP, D), jnp.float32),
                  jax.ShapeDtypeStruct((NCH * NP, CW), jnp.float32),
                  jax.ShapeDtypeStruct((NCH * NP, CW), jnp.float32)),
        mesh=mesh,
        compiler_params=params,
        scratch_types=(
            [pltpu.VMEM_SHARED((NP, CW), jnp.float32)] +
            [pltpu.VMEM((SB, 2 * CW), jnp.float32)] * 2 +
            [pltpu.VMEM((128,), jnp.int32)] * 6 +
            [pltpu.VMEM((128, CW), jnp.float32)] * 3 +
            [pltpu.SemaphoreType.DMA] * 10))
    sc2 = pl.kernel(
        _sc2_body,
        out_type=jax.ShapeDtypeStruct((2 * NP, HID), jnp.float32),
        mesh=mesh,
        compiler_params=params,
        scratch_types=[
            pltpu.VMEM_SHARED((NP, HID), jnp.float32),
            pltpu.VMEM((128,), jnp.int32),
            pltpu.VMEM((128,), jnp.int32),
            pltpu.VMEM((128, HID), jnp.float32),
            pltpu.VMEM((128, HID), jnp.float32),
            pltpu.VMEM((128, HID), jnp.float32),
            pltpu.SemaphoreType.DMA,
            pltpu.SemaphoreType.DMA,
        ])
    return sc1, sc2


def kernel(H, edge_nodes, W1, b1, W2, b2):
    f32 = jnp.float32
    sc1, sc2 = _sc_kernels()
    ne = edge_nodes.shape[0]
    en = jnp.concatenate(
        [edge_nodes.astype(jnp.int32),
         jnp.full((NEP - ne, K), N, jnp.int32)], axis=0).reshape(-1)

    nrb = NP // BR
    l1, _hc, _hp = sc1(H.astype(f32), en)

    h1p, h1ch = pl.pallas_call(
        _mid_body,
        grid=(nrb,),
        in_specs=[pl.BlockSpec((BR, D), lambda i: (i, 0)),
                  pl.BlockSpec((D, HID), lambda i: (0, 0)),
                  pl.BlockSpec((1, HID), lambda i: (0, 0))],
        out_specs=[pl.BlockSpec((BR, HID), lambda i: (i, 0))] * 2,
        out_shape=[jax.ShapeDtypeStruct((NP, HID), f32)] * 2,
    )(l1, W1.astype(f32), b1.astype(f32).reshape(1, HID))

    l2 = sc2(h1p, h1ch, en)

    npb = NP // BF
    out = pl.pallas_call(
        _fin_body,
        grid=(N // BF,),
        in_specs=[pl.BlockSpec((BF, HID), lambda i, c=c: (c * npb + i, 0))
                  for c in range(2)] +
                 [pl.BlockSpec((HID, C), lambda i: (0, 0)),
                  pl.BlockSpec((1, C), lambda i: (0, 0))],
        out_specs=pl.BlockSpec((BF, C), lambda i: (i, 0)),
        out_shape=jax.ShapeDtypeStruct((N, C), f32),
    )(l2, l2, W2.astype(f32), b2.astype(f32).reshape(1, C))
    return out


# speedup vs baseline: 1.4253x; 1.0328x over previous
import functools

import jax
import jax.numpy as jnp
from jax import lax
from jax.experimental import pallas as pl
from jax.experimental.pallas import tpu as pltpu
from jax.experimental.pallas import tpu_sc as plsc

N = 50000
D = 128
K = 16
HID = 16
C = 40
NP = 50400
RT = NP // 16
CW = 32
NCH = D // CW
NEP = 25088
EB = 8
NB = NEP // EB
PB1 = NB // 16
PB2 = NB // 32
INV_KM1 = 1.0 / (K - 1)
BR = 1008
BF = 400


def _nsqrt(x):
    xi = plsc.bitcast(x, jnp.int32)
    y = plsc.bitcast(jnp.int32(0x5F3759DF) - (xi >> 1), jnp.float32)
    xh = 0.5 * x
    y = y * (1.5 - xh * y * y)
    return x * y


SB = 63
NSB = RT // SB


def _sc1_body(H, en, out, hc, hp, acc, hbuf_a, hbuf_b,
              idxr_q0, idxr_q1, idxr_q2, idxr_q3,
              idxo_a, idxo_b, rows_a, rows_b, contrib,
              psem_a, psem_b, osem_a, osem_b,
              sem_i0, sem_i1, sem_i2, sem_i3, sem_a, sem_b):
    c = lax.axis_index("c")
    s = lax.axis_index("s")

    z16 = jnp.zeros((16,), jnp.float32)

    col0 = c * (2 * CW)
    row_t = s * RT

    def p0_start(i, buf, sem):
        @pl.when(i < NSB)
        def _():
            r0 = jnp.minimum(row_t + i * SB, N - SB)
            pltpu.async_copy(H.at[pl.ds(r0, SB), pl.ds(col0, 2 * CW)], buf, sem)

    def p0_outs(i, buf):
        r0 = jnp.minimum(row_t + i * SB, N - SB)
        for j in range(2):
            chunk = c * 2 + j
            src = buf.at[pl.ds(0, SB), pl.ds(j * CW, CW)]
            yield (src, hc.at[pl.ds(chunk * NP + r0, SB)])
            yield (src, hp.at[pl.ds(chunk * NP + r0, SB)])

    def p0_proc(i, buf, psem, osem):
        pltpu.make_async_copy(H.at[pl.ds(0, SB), pl.ds(col0, 2 * CW)],
                              buf, psem).wait()
        @pl.when(i >= 2)
        def _():
            for src, dst in list(p0_outs(i, buf))[1::2]:
                pltpu.make_async_copy(src, dst, osem).wait()

        def rowclip(r, _):
            for v in range(4):
                sl = pl.ds(v * 16, 16)
                buf[r, sl] = jnp.clip(buf[r, sl], 1e-7, 10.0)
            return 0

        lax.fori_loop(0, SB, rowclip, 0)
        srcdst = list(p0_outs(i, buf))
        for src, dst in srcdst[0::2]:
            pltpu.async_copy(src, dst, osem)
        for src, dst in srcdst[0::2]:
            pltpu.make_async_copy(src, dst, osem).wait()

        def rowsq(r, _):
            for v in range(4):
                sl = pl.ds(v * 16, 16)
                x = buf[r, sl]
                buf[r, sl] = x * x
            return 0

        lax.fori_loop(0, SB, rowsq, 0)
        for src, dst in srcdst[1::2]:
            pltpu.async_copy(src, dst, osem)

    p0_start(0, hbuf_a, psem_a)

    def p0_pair(it, _):
        i = it * 2
        p0_start(i + 1, hbuf_b, psem_b)
        p0_proc(i, hbuf_a, psem_a, osem_a)
        p0_start(i + 2, hbuf_a, psem_a)
        p0_proc(i + 1, hbuf_b, psem_b, osem_b)
        return 0

    lax.fori_loop(0, NSB // 2, p0_pair, 0)
    for buf, osem, i in ((hbuf_a, osem_a, NSB - 2), (hbuf_b, osem_b, NSB - 1)):
        for src, dst in list(p0_outs(i, buf))[1::2]:
            pltpu.make_async_copy(src, dst, osem).wait()
    plsc.subcore_barrier()

    def idx_start(i, q, isem):
        b = i * 16 + s
        pltpu.async_copy(en.at[pl.ds(b * 128, 128)], q, isem)

    def mk_off(q, isem, idxo, base):
        pltpu.make_async_copy(en.at[pl.ds(0, 128)], q, isem).wait()
        for v in range(8):
            sl = pl.ds(v * 16, 16)
            idxo[sl] = q[sl] + base

    def compute_scatter(idxr, rows):
        for e in range(EB):
            r0 = e * K

            def ksum(kk, tt):
                return (tt[0] + rows[r0 + kk, pl.ds(0, 16)],
                        tt[1] + rows[r0 + kk, pl.ds(16, 16)])

            t0, t1 = lax.fori_loop(0, K, ksum, (z16, z16))

            def kcon(kk, _):
                contrib[r0 + kk, pl.ds(0, 16)] = _nsqrt(
                    (t0 - rows[r0 + kk, pl.ds(0, 16)]) * INV_KM1)
                contrib[r0 + kk, pl.ds(16, 16)] = _nsqrt(
                    (t1 - rows[r0 + kk, pl.ds(16, 16)]) * INV_KM1)
                return 0

            lax.fori_loop(0, K, kcon, 0)
        pltpu.sync_copy(contrib, acc.at[idxr], add=True)

    def fetch(i, idxr, isem, idxo, rows, sem, base):
        idx_start(i, idxr, isem)
        mk_off(idxr, isem, idxo, base)
        pltpu.async_copy(hp.at[idxo], rows, sem)

    def chunk_pass(j, _):
        chunk = c * 2 + j
        base = chunk * NP
        pltpu.sync_copy(hc.at[pl.ds(base + s * RT, RT)], acc.at[pl.ds(s * RT, RT)])
        fetch(0, idxr_q0, sem_i0, idxo_a, rows_a, sem_a, base)
        plsc.subcore_barrier()

        def pair(it, _):
            i = it * 2
            fetch(i + 1, idxr_q1, sem_i1, idxo_b, rows_b, sem_b, base)
            pltpu.make_async_copy(hp.at[idxo_a], rows_a, sem_a).wait()
            compute_scatter(idxr_q0, rows_a)

            @pl.when(it + 1 < PB1 // 2)
            def _():
                fetch(i + 2, idxr_q0, sem_i0, idxo_a, rows_a, sem_a, base)

            pltpu.make_async_copy(hp.at[idxo_b], rows_b, sem_b).wait()
            compute_scatter(idxr_q1, rows_b)
            return 0

        lax.fori_loop(0, PB1 // 2, pair, 0)
        plsc.subcore_barrier()
        pltpu.sync_copy(acc.at[pl.ds(s * RT, RT)],
                        out.at[pl.ds(s * RT, RT), pl.ds(chunk * CW, CW)])
        plsc.subcore_barrier()
        return 0

    lax.fori_loop(0, 2, chunk_pass, 0)


def _sc2_body(h1p, h1ch, en, out, acc, idxr_a, idxr_b, rows_a, rows_b, contrib,
              sem_a, sem_b):
    c = lax.axis_index("c")
    s = lax.axis_index("s")

    z16 = jnp.zeros((16,), jnp.float32)

    def fetch(i, idxr, rows, sem):
        b = c * (NB // 2) + i * 16 + s
        pltpu.sync_copy(en.at[pl.ds(b * 128, 128)], idxr)
        pltpu.async_copy(h1p.at[idxr], rows, sem)

    def compute_scatter(idxr, rows):
        for e in range(EB):
            r0 = e * K

            def ksum(kk, a):
                return a + rows[r0 + kk, :]

            t = lax.fori_loop(0, K, ksum, z16)

            def kcon(kk, _):
                contrib[r0 + kk, :] = _nsqrt((t - rows[r0 + kk, :]) * INV_KM1)
                return 0

            lax.fori_loop(0, K, kcon, 0)
        pltpu.sync_copy(contrib, acc.at[idxr], add=True)

    fetch(0, idxr_a, rows_a, sem_a)
    pltpu.sync_copy(h1ch.at[pl.ds(s * RT, RT)], acc.at[pl.ds(s * RT, RT)])
    plsc.subcore_barrier()

    def pair(it, _):
        i = it * 2
        fetch(i + 1, idxr_b, rows_b, sem_b)
        pltpu.make_async_copy(h1p.at[idxr_a], rows_a, sem_a).wait()
        compute_scatter(idxr_a, rows_a)

        @pl.when(it + 1 < PB2 // 2)
        def _():
            fetch(i + 2, idxr_a, rows_a, sem_a)

        pltpu.make_async_copy(h1p.at[idxr_b], rows_b, sem_b).wait()
        compute_scatter(idxr_b, rows_b)
        return 0

    lax.fori_loop(0, PB2 // 2, pair, 0)
    plsc.subcore_barrier()
    pltpu.sync_copy(acc.at[pl.ds(s * RT, RT)], out.at[pl.ds(c * NP + s * RT, RT)])


def _mid_body(l1_ref, w1_ref, b1_ref, h1p_ref, h1ch_ref):
    x = l1_ref[...]
    rs = jnp.sum(x, axis=1)
    h = jnp.dot(x, w1_ref[...], preferred_element_type=jnp.float32)
    rinv = 1.0 / rs
    rinv = jnp.where(jnp.isinf(rinv), 0.0, rinv)
    h1 = jnp.maximum(h * rinv[:, None] + b1_ref[...], 0.0)
    h1c = jnp.clip(h1, 1e-7, 10.0)
    h1p_ref[...] = h1c * h1c
    h1ch_ref[...] = 0.5 * h1c


def _fin_body(p0_ref, p1_ref, w2_ref, b2_ref, out_ref):
    ah = p0_ref[...] + p1_ref[...]
    rs = jnp.sum(ah, axis=1)
    rinv = 1.0 / rs
    rinv = jnp.where(jnp.isinf(rinv), 0.0, rinv)
    out_ref[...] = (jnp.dot(ah, w2_ref[...], preferred_element_type=jnp.float32)
                    * rinv[:, None] + b2_ref[...])


@functools.lru_cache(maxsize=None)
def _sc_kernels():
    mesh = plsc.VectorSubcoreMesh(
        core_axis_name="c", subcore_axis_name="s", num_cores=2, num_subcores=16)
    params = pltpu.CompilerParams(
        needs_layout_passes=False, use_tc_tiling_on_sc=False)
    sc1 = pl.kernel(
        _sc1_body,
        out_type=(jax.ShapeDtypeStruct((NP, D), jnp.float32),
                  jax.ShapeDtypeStruct((NCH * NP, CW), jnp.float32),
                  jax.ShapeDtypeStruct((NCH * NP, CW), jnp.float32)),
        mesh=mesh,
        compiler_params=params,
        scratch_types=(
            [pltpu.VMEM_SHARED((NP, CW), jnp.float32)] +
            [pltpu.VMEM((SB, 2 * CW), jnp.float32)] * 2 +
            [pltpu.VMEM((128,), jnp.int32)] * 6 +
            [pltpu.VMEM((128, CW), jnp.float32)] * 3 +
            [pltpu.SemaphoreType.DMA] * 10))
    sc2 = pl.kernel(
        _sc2_body,
        out_type=jax.ShapeDtypeStruct((2 * NP, HID), jnp.float32),
        mesh=mesh,
        compiler_params=params,
        scratch_types=[
            pltpu.VMEM_SHARED((NP, HID), jnp.float32),
            pltpu.VMEM((128,), jnp.int32),
            pltpu.VMEM((128,), jnp.int32),
            pltpu.VMEM((128, HID), jnp.float32),
            pltpu.VMEM((128, HID), jnp.float32),
            pltpu.VMEM((128, HID), jnp.float32),
            pltpu.SemaphoreType.DMA,
            pltpu.SemaphoreType.DMA,
        ])
    return sc1, sc2


def kernel(H, edge_nodes, W1, b1, W2, b2):
    f32 = jnp.float32
    sc1, sc2 = _sc_kernels()
    ne = edge_nodes.shape[0]
    en = jnp.concatenate(
        [edge_nodes.astype(jnp.int32).reshape(-1),
         jnp.full(((NEP - ne) * K,), N, jnp.int32)])

    nrb = NP // BR
    l1, _hc, _hp = sc1(H.astype(f32), en)

    h1p, h1ch = pl.pallas_call(
        _mid_body,
        grid=(nrb,),
        in_specs=[pl.BlockSpec((BR, D), lambda i: (i, 0)),
                  pl.BlockSpec((D, HID), lambda i: (0, 0)),
                  pl.BlockSpec((1, HID), lambda i: (0, 0))],
        out_specs=[pl.BlockSpec((BR, HID), lambda i: (i, 0))] * 2,
        out_shape=[jax.ShapeDtypeStruct((NP, HID), f32)] * 2,
    )(l1, W1.astype(f32), b1.astype(f32).reshape(1, HID))

    l2 = sc2(h1p, h1ch, en)

    npb = NP // BF
    out = pl.pallas_call(
        _fin_body,
        grid=(N // BF,),
        in_specs=[pl.BlockSpec((BF, HID), lambda i, c=c: (c * npb + i, 0))
                  for c in range(2)] +
                 [pl.BlockSpec((HID, C), lambda i: (0, 0)),
                  pl.BlockSpec((1, C), lambda i: (0, 0))],
        out_specs=pl.BlockSpec((BF, C), lambda i: (i, 0)),
        out_shape=jax.ShapeDtypeStruct((N, C), f32),
    )(l2, l2, W2.astype(f32), b2.astype(f32).reshape(1, C))
    return out


# speedup vs baseline: 1.4708x; 1.0319x over previous
import functools

import jax
import jax.numpy as jnp
from jax import lax
from jax.experimental import pallas as pl
from jax.experimental.pallas import tpu as pltpu
from jax.experimental.pallas import tpu_sc as plsc

N = 50000
D = 128
K = 16
HID = 16
C = 40
NP = 50400
RT = NP // 16
CW = 32
NCH = D // CW
NEP = 25088
EB = 8
NB = NEP // EB
PB1 = NB // 16
PB2 = NB // 32
INV_KM1 = 1.0 / (K - 1)
BR = 1008
BF = 400


def _nsqrt(x):
    xi = plsc.bitcast(x, jnp.int32)
    y = plsc.bitcast(jnp.int32(0x5F3759DF) - (xi >> 1), jnp.float32)
    xh = 0.5 * x
    y = y * (1.5 - xh * y * y)
    return x * y


SB = 63
NSB = RT // SB


def _sc1_body(H, en, out, hc, hp, acc, hbuf_a, hbuf_b,
              idxr_q0, idxr_q1, idxr_q2, idxr_q3,
              idxo_a, idxo_b, rows_a, rows_b, contrib,
              psem_a, psem_b, osem_a, osem_b,
              sem_i0, sem_i1, sem_i2, sem_i3, sem_a, sem_b):
    c = lax.axis_index("c")
    s = lax.axis_index("s")

    z16 = jnp.zeros((16,), jnp.float32)

    col0 = c * (2 * CW)
    row_t = s * RT

    def p0_start(i, buf, sem):
        @pl.when(i < NSB)
        def _():
            r0 = jnp.minimum(row_t + i * SB, N - SB)
            pltpu.async_copy(H.at[pl.ds(r0, SB), pl.ds(col0, 2 * CW)], buf, sem)

    def p0_outs(i, buf):
        r0 = jnp.minimum(row_t + i * SB, N - SB)
        for j in range(2):
            chunk = c * 2 + j
            src = buf.at[pl.ds(0, SB), pl.ds(j * CW, CW)]
            yield (src, hc.at[pl.ds(chunk * NP + r0, SB)])
            yield (src, hp.at[pl.ds(chunk * NP + r0, SB)])

    def p0_proc(i, buf, psem, osem):
        pltpu.make_async_copy(H.at[pl.ds(0, SB), pl.ds(col0, 2 * CW)],
                              buf, psem).wait()
        @pl.when(i >= 2)
        def _():
            for src, dst in list(p0_outs(i, buf))[1::2]:
                pltpu.make_async_copy(src, dst, osem).wait()

        def rowclip(r, _):
            for v in range(4):
                sl = pl.ds(v * 16, 16)
                buf[r, sl] = jnp.clip(buf[r, sl], 1e-7, 10.0)
            return 0

        lax.fori_loop(0, SB, rowclip, 0)
        srcdst = list(p0_outs(i, buf))
        for src, dst in srcdst[0::2]:
            pltpu.async_copy(src, dst, osem)
        for src, dst in srcdst[0::2]:
            pltpu.make_async_copy(src, dst, osem).wait()

        def rowsq(r, _):
            for v in range(4):
                sl = pl.ds(v * 16, 16)
                x = buf[r, sl]
                buf[r, sl] = x * x
            return 0

        lax.fori_loop(0, SB, rowsq, 0)
        for src, dst in srcdst[1::2]:
            pltpu.async_copy(src, dst, osem)

    p0_start(0, hbuf_a, psem_a)

    def p0_pair(it, _):
        i = it * 2
        p0_start(i + 1, hbuf_b, psem_b)
        p0_proc(i, hbuf_a, psem_a, osem_a)
        p0_start(i + 2, hbuf_a, psem_a)
        p0_proc(i + 1, hbuf_b, psem_b, osem_b)
        return 0

    lax.fori_loop(0, NSB // 2, p0_pair, 0)
    for buf, osem, i in ((hbuf_a, osem_a, NSB - 2), (hbuf_b, osem_b, NSB - 1)):
        for src, dst in list(p0_outs(i, buf))[1::2]:
            pltpu.make_async_copy(src, dst, osem).wait()
    plsc.subcore_barrier()

    def idx_start(i, q, isem):
        b = i * 16 + s
        pltpu.async_copy(en.at[pl.ds(b * 128, 128)], q, isem)

    def mk_off(q, isem, idxo, base):
        pltpu.make_async_copy(en.at[pl.ds(0, 128)], q, isem).wait()
        for v in range(8):
            sl = pl.ds(v * 16, 16)
            idxo[sl] = q[sl] + base

    def compute_scatter(idxr, rows):
        for e in range(EB):
            r0 = e * K

            def ksum(kk, tt):
                return (tt[0] + rows[r0 + kk, pl.ds(0, 16)],
                        tt[1] + rows[r0 + kk, pl.ds(16, 16)])

            t0, t1 = lax.fori_loop(0, K, ksum, (z16, z16))

            def kcon(kk, _):
                contrib[r0 + kk, pl.ds(0, 16)] = _nsqrt(
                    (t0 - rows[r0 + kk, pl.ds(0, 16)]) * INV_KM1)
                contrib[r0 + kk, pl.ds(16, 16)] = _nsqrt(
                    (t1 - rows[r0 + kk, pl.ds(16, 16)]) * INV_KM1)
                return 0

            lax.fori_loop(0, K, kcon, 0)
        pltpu.sync_copy(contrib, acc.at[idxr], add=True)

    qs = (idxr_q0, idxr_q1, idxr_q2, idxr_q3)
    qsem = (sem_i0, sem_i1, sem_i2, sem_i3)
    gbuf = (rows_a, rows_b)
    gsem = (sem_a, sem_b)
    gidxo = (idxo_a, idxo_b)

    def chunk_pass(j, _):
        chunk = c * 2 + j
        base = chunk * NP
        for k in range(4):
            idx_start(k, qs[k], qsem[k])
        pltpu.sync_copy(hc.at[pl.ds(base + s * RT, RT)], acc.at[pl.ds(s * RT, RT)])
        for k in range(2):
            mk_off(qs[k], qsem[k], gidxo[k], base)
            pltpu.async_copy(hp.at[gidxo[k]], gbuf[k], gsem[k])
        plsc.subcore_barrier()

        def quad(it, _):
            i = it * 4
            for k in range(4):
                g = k % 2
                pltpu.make_async_copy(hp.at[gidxo[g]], gbuf[g], gsem[g]).wait()
                compute_scatter(qs[k], gbuf[g])

                @pl.when(i + 4 + k < PB1)
                def _():
                    idx_start(i + 4 + k, qs[k], qsem[k])

                @pl.when(i + 2 + k < PB1)
                def _():
                    kn = (k + 2) % 4
                    mk_off(qs[kn], qsem[kn], gidxo[g], base)
                    pltpu.async_copy(hp.at[gidxo[g]], gbuf[g], gsem[g])
            return 0

        lax.fori_loop(0, PB1 // 4, quad, 0)
        plsc.subcore_barrier()
        pltpu.sync_copy(acc.at[pl.ds(s * RT, RT)],
                        out.at[pl.ds(s * RT, RT), pl.ds(chunk * CW, CW)])
        plsc.subcore_barrier()
        return 0

    lax.fori_loop(0, 2, chunk_pass, 0)


def _sc2_body(h1p, h1ch, en, out, acc, idxr_a, idxr_b, rows_a, rows_b, contrib,
              sem_a, sem_b):
    c = lax.axis_index("c")
    s = lax.axis_index("s")

    z16 = jnp.zeros((16,), jnp.float32)

    def fetch(i, idxr, rows, sem):
        b = c * (NB // 2) + i * 16 + s
        pltpu.sync_copy(en.at[pl.ds(b * 128, 128)], idxr)
        pltpu.async_copy(h1p.at[idxr], rows, sem)

    def compute_scatter(idxr, rows):
        for e in range(EB):
            r0 = e * K

            def ksum(kk, a):
                return a + rows[r0 + kk, :]

            t = lax.fori_loop(0, K, ksum, z16)

            def kcon(kk, _):
                contrib[r0 + kk, :] = _nsqrt((t - rows[r0 + kk, :]) * INV_KM1)
                return 0

            lax.fori_loop(0, K, kcon, 0)
        pltpu.sync_copy(contrib, acc.at[idxr], add=True)

    fetch(0, idxr_a, rows_a, sem_a)
    pltpu.sync_copy(h1ch.at[pl.ds(s * RT, RT)], acc.at[pl.ds(s * RT, RT)])
    plsc.subcore_barrier()

    def pair(it, _):
        i = it * 2
        fetch(i + 1, idxr_b, rows_b, sem_b)
        pltpu.make_async_copy(h1p.at[idxr_a], rows_a, sem_a).wait()
        compute_scatter(idxr_a, rows_a)

        @pl.when(it + 1 < PB2 // 2)
        def _():
            fetch(i + 2, idxr_a, rows_a, sem_a)

        pltpu.make_async_copy(h1p.at[idxr_b], rows_b, sem_b).wait()
        compute_scatter(idxr_b, rows_b)
        return 0

    lax.fori_loop(0, PB2 // 2, pair, 0)
    plsc.subcore_barrier()
    pltpu.sync_copy(acc.at[pl.ds(s * RT, RT)], out.at[pl.ds(c * NP + s * RT, RT)])


def _mid_body(l1_ref, w1_ref, b1_ref, h1p_ref, h1ch_ref):
    x = l1_ref[...]
    rs = jnp.sum(x, axis=1)
    h = jnp.dot(x, w1_ref[...], preferred_element_type=jnp.float32)
    rinv = 1.0 / rs
    rinv = jnp.where(jnp.isinf(rinv), 0.0, rinv)
    h1 = jnp.maximum(h * rinv[:, None] + b1_ref[...], 0.0)
    h1c = jnp.clip(h1, 1e-7, 10.0)
    h1p_ref[...] = h1c * h1c
    h1ch_ref[...] = 0.5 * h1c


def _fin_body(p0_ref, p1_ref, w2_ref, b2_ref, out_ref):
    ah = p0_ref[...] + p1_ref[...]
    rs = jnp.sum(ah, axis=1)
    rinv = 1.0 / rs
    rinv = jnp.where(jnp.isinf(rinv), 0.0, rinv)
    out_ref[...] = (jnp.dot(ah, w2_ref[...], preferred_element_type=jnp.float32)
                    * rinv[:, None] + b2_ref[...])


@functools.lru_cache(maxsize=None)
def _sc_kernels():
    mesh = plsc.VectorSubcoreMesh(
        core_axis_name="c", subcore_axis_name="s", num_cores=2, num_subcores=16)
    params = pltpu.CompilerParams(
        needs_layout_passes=False, use_tc_tiling_on_sc=False)
    sc1 = pl.kernel(
        _sc1_body,
        out_type=(jax.ShapeDtypeStruct((NP, D), jnp.float32),
                  jax.ShapeDtypeStruct((NCH * NP, CW), jnp.float32),
                  jax.ShapeDtypeStruct((NCH * NP, CW), jnp.float32)),
        mesh=mesh,
        compiler_params=params,
        scratch_types=(
            [pltpu.VMEM_SHARED((NP, CW), jnp.float32)] +
            [pltpu.VMEM((SB, 2 * CW), jnp.float32)] * 2 +
            [pltpu.VMEM((128,), jnp.int32)] * 6 +
            [pltpu.VMEM((128, CW), jnp.float32)] * 3 +
            [pltpu.SemaphoreType.DMA] * 10))
    sc2 = pl.kernel(
        _sc2_body,
        out_type=jax.ShapeDtypeStruct((2 * NP, HID), jnp.float32),
        mesh=mesh,
        compiler_params=params,
        scratch_types=[
            pltpu.VMEM_SHARED((NP, HID), jnp.float32),
            pltpu.VMEM((128,), jnp.int32),
            pltpu.VMEM((128,), jnp.int32),
            pltpu.VMEM((128, HID), jnp.float32),
            pltpu.VMEM((128, HID), jnp.float32),
            pltpu.VMEM((128, HID), jnp.float32),
            pltpu.SemaphoreType.DMA,
            pltpu.SemaphoreType.DMA,
        ])
    return sc1, sc2


def kernel(H, edge_nodes, W1, b1, W2, b2):
    f32 = jnp.float32
    sc1, sc2 = _sc_kernels()
    ne = edge_nodes.shape[0]
    en = jnp.concatenate(
        [edge_nodes.astype(jnp.int32).reshape(-1),
         jnp.full(((NEP - ne) * K,), N, jnp.int32)])

    nrb = NP // BR
    l1, _hc, _hp = sc1(H.astype(f32), en)

    h1p, h1ch = pl.pallas_call(
        _mid_body,
        grid=(nrb,),
        in_specs=[pl.BlockSpec((BR, D), lambda i: (i, 0)),
                  pl.BlockSpec((D, HID), lambda i: (0, 0)),
                  pl.BlockSpec((1, HID), lambda i: (0, 0))],
        out_specs=[pl.BlockSpec((BR, HID), lambda i: (i, 0))] * 2,
        out_shape=[jax.ShapeDtypeStruct((NP, HID), f32)] * 2,
    )(l1, W1.astype(f32), b1.astype(f32).reshape(1, HID))

    l2 = sc2(h1p, h1ch, en)

    npb = NP // BF
    out = pl.pallas_call(
        _fin_body,
        grid=(N // BF,),
        in_specs=[pl.BlockSpec((BF, HID), lambda i, c=c: (c * npb + i, 0))
                  for c in range(2)] +
                 [pl.BlockSpec((HID, C), lambda i: (0, 0)),
                  pl.BlockSpec((1, C), lambda i: (0, 0))],
        out_specs=pl.BlockSpec((BF, C), lambda i: (i, 0)),
        out_shape=jax.ShapeDtypeStruct((N, C), f32),
    )(l2, l2, W2.astype(f32), b2.astype(f32).reshape(1, C))
    return out


# speedup vs baseline: 1.5402x; 1.0472x over previous
import functools

import jax
import jax.numpy as jnp
from jax import lax
from jax.experimental import pallas as pl
from jax.experimental.pallas import tpu as pltpu
from jax.experimental.pallas import tpu_sc as plsc

N = 50000
D = 128
K = 16
HID = 16
C = 40
NP = 50400
RT = NP // 16
CW = 32
NCH = D // CW
NEP = 25088
EB = 8
NB = NEP // EB
PB1 = NB // 16
PB2 = NB // 32
INV_KM1 = 1.0 / (K - 1)
BR = 1008
BF = 2000


def _nsqrt(x):
    xi = plsc.bitcast(x, jnp.int32)
    y = plsc.bitcast(jnp.int32(0x5F3759DF) - (xi >> 1), jnp.float32)
    xh = 0.5 * x
    y = y * (1.5 - xh * y * y)
    return x * y


SB = 63
NSB = RT // SB


def _sc1_body(H, en, out, hc, hp, acc, hbuf_a, hbuf_b,
              idxr_q0, idxr_q1, idxr_q2, idxr_q3,
              idxo_a, idxo_b, rows_a, rows_b, contrib,
              psem_a, psem_b, osem_a, osem_b,
              sem_i0, sem_i1, sem_i2, sem_i3, sem_a, sem_b):
    c = lax.axis_index("c")
    s = lax.axis_index("s")

    z16 = jnp.zeros((16,), jnp.float32)

    col0 = c * (2 * CW)
    row_t = s * RT

    def p0_start(i, buf, sem):
        @pl.when(i < NSB)
        def _():
            r0 = jnp.minimum(row_t + i * SB, N - SB)
            pltpu.async_copy(H.at[pl.ds(r0, SB), pl.ds(col0, 2 * CW)], buf, sem)

    def p0_outs(i, buf):
        r0 = jnp.minimum(row_t + i * SB, N - SB)
        for j in range(2):
            chunk = c * 2 + j
            src = buf.at[pl.ds(0, SB), pl.ds(j * CW, CW)]
            yield (src, hc.at[pl.ds(chunk * NP + r0, SB)])
            yield (src, hp.at[pl.ds(chunk * NP + r0, SB)])

    def p0_proc(i, buf, psem, osem):
        pltpu.make_async_copy(H.at[pl.ds(0, SB), pl.ds(col0, 2 * CW)],
                              buf, psem).wait()
        @pl.when(i >= 2)
        def _():
            for src, dst in list(p0_outs(i, buf))[1::2]:
                pltpu.make_async_copy(src, dst, osem).wait()

        def rowclip(r, _):
            for v in range(4):
                sl = pl.ds(v * 16, 16)
                buf[r, sl] = jnp.clip(buf[r, sl], 1e-7, 10.0)
            return 0

        lax.fori_loop(0, SB, rowclip, 0)
        srcdst = list(p0_outs(i, buf))
        for src, dst in srcdst[0::2]:
            pltpu.async_copy(src, dst, osem)
        for src, dst in srcdst[0::2]:
            pltpu.make_async_copy(src, dst, osem).wait()

        def rowsq(r, _):
            for v in range(4):
                sl = pl.ds(v * 16, 16)
                x = buf[r, sl]
                buf[r, sl] = x * x
            return 0

        lax.fori_loop(0, SB, rowsq, 0)
        for src, dst in srcdst[1::2]:
            pltpu.async_copy(src, dst, osem)

    p0_start(0, hbuf_a, psem_a)

    def p0_pair(it, _):
        i = it * 2
        p0_start(i + 1, hbuf_b, psem_b)
        p0_proc(i, hbuf_a, psem_a, osem_a)
        p0_start(i + 2, hbuf_a, psem_a)
        p0_proc(i + 1, hbuf_b, psem_b, osem_b)
        return 0

    lax.fori_loop(0, NSB // 2, p0_pair, 0)
    for buf, osem, i in ((hbuf_a, osem_a, NSB - 2), (hbuf_b, osem_b, NSB - 1)):
        for src, dst in list(p0_outs(i, buf))[1::2]:
            pltpu.make_async_copy(src, dst, osem).wait()
    plsc.subcore_barrier()

    def idx_start(i, q, isem):
        b = i * 16 + s
        pltpu.async_copy(en.at[pl.ds(b * 128, 128)], q, isem)

    def mk_off(q, isem, idxo, base):
        pltpu.make_async_copy(en.at[pl.ds(0, 128)], q, isem).wait()
        for v in range(8):
            sl = pl.ds(v * 16, 16)
            idxo[sl] = q[sl] + base

    def compute_scatter(idxr, rows):
        for e in range(EB):
            r0 = e * K

            def ksum(kk, tt):
                return (tt[0] + rows[r0 + kk, pl.ds(0, 16)],
                        tt[1] + rows[r0 + kk, pl.ds(16, 16)])

            t0, t1 = lax.fori_loop(0, K, ksum, (z16, z16))

            def kcon(kk, _):
                contrib[r0 + kk, pl.ds(0, 16)] = _nsqrt(
                    (t0 - rows[r0 + kk, pl.ds(0, 16)]) * INV_KM1)
                contrib[r0 + kk, pl.ds(16, 16)] = _nsqrt(
                    (t1 - rows[r0 + kk, pl.ds(16, 16)]) * INV_KM1)
                return 0

            lax.fori_loop(0, K, kcon, 0)
        pltpu.sync_copy(contrib, acc.at[idxr], add=True)

    qs = (idxr_q0, idxr_q1, idxr_q2, idxr_q3)
    qsem = (sem_i0, sem_i1, sem_i2, sem_i3)
    gbuf = (rows_a, rows_b)
    gsem = (sem_a, sem_b)
    gidxo = (idxo_a, idxo_b)

    def chunk_pass(j, _):
        chunk = c * 2 + j
        base = chunk * NP
        for k in range(4):
            idx_start(k, qs[k], qsem[k])
        pltpu.sync_copy(hc.at[pl.ds(base + s * RT, RT)], acc.at[pl.ds(s * RT, RT)])
        for k in range(2):
            mk_off(qs[k], qsem[k], gidxo[k], base)
            pltpu.async_copy(hp.at[gidxo[k]], gbuf[k], gsem[k])
        plsc.subcore_barrier()

        def quad(it, _):
            i = it * 4
            for k in range(4):
                g = k % 2
                pltpu.make_async_copy(hp.at[gidxo[g]], gbuf[g], gsem[g]).wait()
                compute_scatter(qs[k], gbuf[g])

                @pl.when(i + 4 + k < PB1)
                def _():
                    idx_start(i + 4 + k, qs[k], qsem[k])

                @pl.when(i + 2 + k < PB1)
                def _():
                    kn = (k + 2) % 4
                    mk_off(qs[kn], qsem[kn], gidxo[g], base)
                    pltpu.async_copy(hp.at[gidxo[g]], gbuf[g], gsem[g])
            return 0

        lax.fori_loop(0, PB1 // 4, quad, 0)
        plsc.subcore_barrier()
        pltpu.sync_copy(acc.at[pl.ds(s * RT, RT)],
                        out.at[pl.ds(s * RT, RT), pl.ds(chunk * CW, CW)])
        plsc.subcore_barrier()
        return 0

    lax.fori_loop(0, 2, chunk_pass, 0)


def _sc2_body(h1p, h1ch, en, out_a, out_b, acc, idxr_a, idxr_b, rows_a, rows_b,
              contrib, sem_a, sem_b):
    c = lax.axis_index("c")
    s = lax.axis_index("s")

    z16 = jnp.zeros((16,), jnp.float32)

    def fetch(i, idxr, rows, sem):
        b = c * (NB // 2) + i * 16 + s
        pltpu.sync_copy(en.at[pl.ds(b * 128, 128)], idxr)
        pltpu.async_copy(h1p.at[idxr], rows, sem)

    def compute_scatter(idxr, rows):
        for e in range(EB):
            r0 = e * K

            def ksum(kk, a):
                return a + rows[r0 + kk, :]

            t = lax.fori_loop(0, K, ksum, z16)

            def kcon(kk, _):
                contrib[r0 + kk, :] = _nsqrt((t - rows[r0 + kk, :]) * INV_KM1)
                return 0

            lax.fori_loop(0, K, kcon, 0)
        pltpu.sync_copy(contrib, acc.at[idxr], add=True)

    fetch(0, idxr_a, rows_a, sem_a)
    pltpu.sync_copy(h1ch.at[pl.ds(s * RT, RT)], acc.at[pl.ds(s * RT, RT)])
    plsc.subcore_barrier()

    def pair(it, _):
        i = it * 2
        fetch(i + 1, idxr_b, rows_b, sem_b)
        pltpu.make_async_copy(h1p.at[idxr_a], rows_a, sem_a).wait()
        compute_scatter(idxr_a, rows_a)

        @pl.when(it + 1 < PB2 // 2)
        def _():
            fetch(i + 2, idxr_a, rows_a, sem_a)

        pltpu.make_async_copy(h1p.at[idxr_b], rows_b, sem_b).wait()
        compute_scatter(idxr_b, rows_b)
        return 0

    lax.fori_loop(0, PB2 // 2, pair, 0)
    plsc.subcore_barrier()

    @pl.when(c == 0)
    def _():
        pltpu.sync_copy(acc.at[pl.ds(s * RT, RT)], out_a.at[pl.ds(s * RT, RT)])

    @pl.when(c == 1)
    def _():
        pltpu.sync_copy(acc.at[pl.ds(s * RT, RT)], out_b.at[pl.ds(s * RT, RT)])


def _mid_body(l1_ref, w1_ref, b1_ref, h1p_ref, h1ch_ref):
    x = l1_ref[...]
    rs = jnp.sum(x, axis=1)
    h = jnp.dot(x, w1_ref[...], preferred_element_type=jnp.float32)
    rinv = 1.0 / rs
    rinv = jnp.where(jnp.isinf(rinv), 0.0, rinv)
    h1 = jnp.maximum(h * rinv[:, None] + b1_ref[...], 0.0)
    h1c = jnp.clip(h1, 1e-7, 10.0)
    h1p_ref[...] = h1c * h1c
    h1ch_ref[...] = 0.5 * h1c


def _fin_body(p0_ref, p1_ref, w2_ref, b2_ref, out_ref):
    ah = p0_ref[...] + p1_ref[...]
    rs = jnp.sum(ah, axis=1)
    rinv = 1.0 / rs
    rinv = jnp.where(jnp.isinf(rinv), 0.0, rinv)
    out_ref[...] = (jnp.dot(ah, w2_ref[...], preferred_element_type=jnp.float32)
                    * rinv[:, None] + b2_ref[...])


@functools.lru_cache(maxsize=None)
def _sc_kernels():
    mesh = plsc.VectorSubcoreMesh(
        core_axis_name="c", subcore_axis_name="s", num_cores=2, num_subcores=16)
    params = pltpu.CompilerParams(
        needs_layout_passes=False, use_tc_tiling_on_sc=False)
    sc1 = pl.kernel(
        _sc1_body,
        out_type=(jax.ShapeDtypeStruct((NP, D), jnp.float32),
                  jax.ShapeDtypeStruct((NCH * NP, CW), jnp.float32),
                  jax.ShapeDtypeStruct((NCH * NP, CW), jnp.float32)),
        mesh=mesh,
        compiler_params=params,
        scratch_types=(
            [pltpu.VMEM_SHARED((NP, CW), jnp.float32)] +
            [pltpu.VMEM((SB, 2 * CW), jnp.float32)] * 2 +
            [pltpu.VMEM((128,), jnp.int32)] * 6 +
            [pltpu.VMEM((128, CW), jnp.float32)] * 3 +
            [pltpu.SemaphoreType.DMA] * 10))
    sc2 = pl.kernel(
        _sc2_body,
        out_type=(jax.ShapeDtypeStruct((NP, HID), jnp.float32),
                  jax.ShapeDtypeStruct((NP, HID), jnp.float32)),
        mesh=mesh,
        compiler_params=params,
        scratch_types=[
            pltpu.VMEM_SHARED((NP, HID), jnp.float32),
            pltpu.VMEM((128,), jnp.int32),
            pltpu.VMEM((128,), jnp.int32),
            pltpu.VMEM((128, HID), jnp.float32),
            pltpu.VMEM((128, HID), jnp.float32),
            pltpu.VMEM((128, HID), jnp.float32),
            pltpu.SemaphoreType.DMA,
            pltpu.SemaphoreType.DMA,
        ])
    return sc1, sc2


def kernel(H, edge_nodes, W1, b1, W2, b2):
    f32 = jnp.float32
    sc1, sc2 = _sc_kernels()
    ne = edge_nodes.shape[0]
    en = jnp.concatenate(
        [edge_nodes.astype(jnp.int32).reshape(-1),
         jnp.full(((NEP - ne) * K,), N, jnp.int32)])

    nrb = NP // BR
    l1, _hc, _hp = sc1(H.astype(f32), en)

    h1p, h1ch = pl.pallas_call(
        _mid_body,
        grid=(nrb,),
        in_specs=[pl.BlockSpec((BR, D), lambda i: (i, 0)),
                  pl.BlockSpec((D, HID), lambda i: (0, 0)),
                  pl.BlockSpec((1, HID), lambda i: (0, 0))],
        out_specs=[pl.BlockSpec((BR, HID), lambda i: (i, 0))] * 2,
        out_shape=[jax.ShapeDtypeStruct((NP, HID), f32)] * 2,
    )(l1, W1.astype(f32), b1.astype(f32).reshape(1, HID))

    l2a, l2b = sc2(h1p, h1ch, en)

    out = pl.pallas_call(
        _fin_body,
        grid=(N // BF,),
        in_specs=[pl.BlockSpec((BF, HID), lambda i: (i, 0)),
                  pl.BlockSpec((BF, HID), lambda i: (i, 0)),
                  pl.BlockSpec((HID, C), lambda i: (0, 0)),
                  pl.BlockSpec((1, C), lambda i: (0, 0))],
        out_specs=pl.BlockSpec((BF, C), lambda i: (i, 0)),
        out_shape=jax.ShapeDtypeStruct((N, C), f32),
    )(l2a, l2b, W2.astype(f32), b2.astype(f32).reshape(1, C))
    return out
